# Initial kernel scaffold; baseline (speedup 1.0000x reference)
import jax
import jax.numpy as jnp
from jax.experimental import pallas as pl


def kernel(im, s, num_clips, num_caps):
    raise NotImplementedError("write your pallas kernel here")


# speedup vs baseline: 29.5899x; 29.5899x over previous
import jax
import jax.numpy as jnp
from jax import lax
from jax.experimental import pallas as pl
from jax.experimental.pallas import tpu as pltpu

_N = 16


def _loss_kernel(starts_r, ends_r, starts_c, ends_c, counts_ref,
                 im_ref, s_ref, out_ref):
    im = im_ref[:, :]
    sm = s_ref[:, :]
    total_clips = im.shape[0]
    total_caps = sm.shape[0]

    r_iota = lax.broadcasted_iota(jnp.int32, (_N, total_clips), 1)
    rmask = ((r_iota >= starts_r[:, :]) & (r_iota < ends_r[:, :])
             ).astype(jnp.float32)
    c_iota = lax.broadcasted_iota(jnp.int32, (_N, total_caps), 1)
    cmask = ((c_iota >= starts_c[:, :]) & (c_iota < ends_c[:, :])
             ).astype(jnp.float32)

    im_g = jnp.dot(rmask, im, preferred_element_type=jnp.float32)
    s_g = jnp.dot(cmask, sm, preferred_element_type=jnp.float32)

    block = jnp.dot(im_g, s_g.T, preferred_element_type=jnp.float32)
    scores_reduced = block / counts_ref[:, :]

    eye = jnp.eye(_N, dtype=bool)
    diag = jnp.sum(jnp.where(eye, scores_reduced, 0.0), axis=1,
                   keepdims=True)
    cost_s = jnp.maximum(scores_reduced - diag, 0.0)
    cost_im = jnp.maximum(scores_reduced - diag.T, 0.0)
    cost_s = jnp.where(eye, 0.0, cost_s)
    cost_im = jnp.where(eye, 0.0, cost_im)
    out_ref[:, :] = jnp.sum(cost_s + cost_im, axis=(0, 1), keepdims=True)


def kernel(im, s, num_clips, num_caps):
    cum_r = jnp.cumsum(num_clips)
    cum_c = jnp.cumsum(num_caps)
    starts_r = (cum_r - num_clips).reshape(_N, 1)
    ends_r = cum_r.reshape(_N, 1)
    starts_c = (cum_c - num_caps).reshape(_N, 1)
    ends_c = cum_c.reshape(_N, 1)
    counts = (num_clips[:, None] * num_caps[None, :]).astype(jnp.float32)

    out = pl.pallas_call(
        _loss_kernel,
        out_shape=jax.ShapeDtypeStruct((1, 1), jnp.float32),
    )(starts_r, ends_r, starts_c, ends_c, counts, im, s)
    return out[0, 0]
